# baseline (speedup 1.0000x reference)
import functools

import jax
import jax.numpy as jnp
from jax import lax
from jax.experimental import pallas as pl
from jax.experimental.pallas import tpu as pltpu
from jax.experimental.pallas import tpu_sc as plsc

_N = 10000
_E = 320000
_G = 64

_NC = 2
_NS = 16
_NPAD = 10112
_RPT = _NPAD // _NS
_CHUNK = 128
_NCHUNKS = _E // _CHUNK
_CH_PER_CORE = _NCHUNKS // _NC
_T_STEPS = -(-_CH_PER_CORE // _NS)

_mesh = plsc.VectorSubcoreMesh(core_axis_name="c", subcore_axis_name="s")

_TMAX = 79
_NCHPAD = 2528
_NB = 4


_DEGW = 16


@functools.partial(
    pl.kernel,
    out_type=jax.ShapeDtypeStruct((_NC, _NPAD, _DEGW), jnp.float32),
    mesh=_mesh,
    scratch_types=[
        pltpu.VMEM_SHARED((_NPAD, _DEGW), jnp.float32),
        pltpu.VMEM((_CHUNK, _DEGW), jnp.float32),
        pltpu.VMEM((_TMAX, _CHUNK), jnp.int32),
        pltpu.SemaphoreType.DMA,
    ],
    name="gcn_deg",
    compiler_params=pltpu.CompilerParams(use_tc_tiling_on_sc=False),
)
def _deg_sc(dst2d_hbm, ones_hbm, zeros_hbm, out_hbm, acc, ones_v, didx2d, sem):
    c = lax.axis_index("c")
    s = lax.axis_index("s")
    w = c * _NS + s
    r0 = s * _RPT
    pltpu.sync_copy(zeros_hbm.at[pl.ds(r0, _RPT)], acc.at[pl.ds(r0, _RPT)])
    pltpu.sync_copy(ones_hbm, ones_v)
    base = 78 * w + jnp.minimum(w, 4)
    cnt = jnp.where(w < 4, 79, 78)
    pltpu.sync_copy(dst2d_hbm.at[pl.ds(base, _TMAX)], didx2d)
    plsc.subcore_barrier()

    def body(grp, carry):
        for b in range(_NB):
            t = grp * _NB + b

            @pl.when(t < cnt)
            def _():
                pltpu.async_copy(ones_v, acc.at[didx2d.at[t]], sem, add=True)

        for b in range(_NB):
            t = grp * _NB + b

            @pl.when(t < cnt)
            def _():
                pltpu.make_async_copy(ones_v, acc.at[didx2d.at[0]], sem).wait()

        return carry

    lax.fori_loop(0, -(-_TMAX // _NB), body, 0)
    plsc.subcore_barrier()
    pltpu.sync_copy(acc.at[pl.ds(r0, _RPT)], out_hbm.at[c, pl.ds(r0, _RPT)])


def _ring(g_hbm, acc, rows, gsem, ssem, sidx2d, didx2d, cnt, nb):
    def issue_gather(b, t):
        pltpu.async_copy(g_hbm.at[sidx2d.at[t]], rows[b], gsem[b])

    def wait_gather(b):
        pltpu.make_async_copy(g_hbm.at[sidx2d.at[0]], rows[b], gsem[b]).wait()

    def issue_scat(b, t):
        pltpu.async_copy(rows[b], acc.at[didx2d.at[t]], ssem[b], add=True)

    def wait_scat(b):
        pltpu.make_async_copy(rows[b], acc.at[didx2d.at[0]], ssem[b]).wait()

    for b in range(nb):
        issue_gather(b, b)

    def body(grp, carry):
        for b in range(nb):
            t = grp * nb + b

            @pl.when(t < cnt)
            def _():
                wait_gather(b)
                issue_scat(b, t)

        for b in range(nb):
            t = grp * nb + b

            @pl.when(t + nb < cnt)
            def _():
                wait_scat(b)
                issue_gather(b, t + nb)

        return carry

    lax.fori_loop(0, -(-_TMAX // nb), body, 0)
    for b in range(nb):
        wait_scat(b)


def _tile_span(w):
    base = 78 * w + jnp.minimum(w, 4)
    cnt = jnp.where(w < 4, 79, 78)
    return base, cnt


_NB1 = 8


@functools.partial(
    pl.kernel,
    out_type=jax.ShapeDtypeStruct((_NC, _NPAD, 64), jnp.float32),
    mesh=_mesh,
    scratch_types=(
        [pltpu.VMEM_SHARED((_NPAD, 64), jnp.float32)]
        + [pltpu.VMEM((_CHUNK, 64), jnp.float32) for _ in range(_NB1)]
        + [pltpu.VMEM((_TMAX, _CHUNK), jnp.int32),
           pltpu.VMEM((_TMAX, _CHUNK), jnp.int32)]
        + [pltpu.SemaphoreType.DMA for _ in range(2 * _NB1)]
    ),
    name="gcn_agg_f64",
    compiler_params=pltpu.CompilerParams(use_tc_tiling_on_sc=False),
)
def _agg64(g_hbm, src2d_hbm, dst2d_hbm, zeros_hbm, out_hbm, acc, *sc):
    rows = list(sc[:_NB1])
    sidx2d, didx2d = sc[_NB1], sc[_NB1 + 1]
    gsem = list(sc[_NB1 + 2:2 * _NB1 + 2])
    ssem = list(sc[2 * _NB1 + 2:])
    c = lax.axis_index("c")
    s = lax.axis_index("s")
    r0 = s * _RPT
    pltpu.sync_copy(zeros_hbm.at[pl.ds(r0, _RPT)], acc.at[pl.ds(r0, _RPT)])
    base, cnt = _tile_span(c * _NS + s)
    pltpu.sync_copy(src2d_hbm.at[pl.ds(base, _TMAX)], sidx2d)
    pltpu.sync_copy(dst2d_hbm.at[pl.ds(base, _TMAX)], didx2d)
    plsc.subcore_barrier()
    _ring(g_hbm, acc, rows, gsem, ssem, sidx2d, didx2d, cnt, _NB1)
    plsc.subcore_barrier()
    pltpu.sync_copy(acc.at[pl.ds(r0, _RPT)], out_hbm.at[c, pl.ds(r0, _RPT)])


_NB2 = 3


@functools.partial(
    pl.kernel,
    out_type=[jax.ShapeDtypeStruct((_NC, _NPAD, 64), jnp.float32),
              jax.ShapeDtypeStruct((_NC, _NPAD, 64), jnp.float32)],
    mesh=_mesh,
    scratch_types=(
        [pltpu.VMEM_SHARED((_NPAD, 64), jnp.float32),
         pltpu.VMEM_SHARED((_NPAD, 64), jnp.float32)]
        + [pltpu.VMEM((_CHUNK, 64), jnp.float32) for _ in range(_NB2)]
        + [pltpu.VMEM((_TMAX, _CHUNK), jnp.int32),
           pltpu.VMEM((_TMAX, _CHUNK), jnp.int32)]
        + [pltpu.SemaphoreType.DMA for _ in range(2 * _NB2)]
    ),
    name="gcn_agg_l1",
    compiler_params=pltpu.CompilerParams(use_tc_tiling_on_sc=False),
)
def _agg_l1(gl_hbm, gr_hbm, src2d_hbm, dst2d_hbm, zeros_hbm,
            outl_hbm, outr_hbm, accl, accr, *sc):
    rows = list(sc[:_NB2])
    sidx2d, didx2d = sc[_NB2], sc[_NB2 + 1]
    gsem = list(sc[_NB2 + 2:2 * _NB2 + 2])
    ssem = list(sc[2 * _NB2 + 2:])
    c = lax.axis_index("c")
    s = lax.axis_index("s")
    r0 = s * _RPT
    pltpu.sync_copy(zeros_hbm.at[pl.ds(r0, _RPT)], accl.at[pl.ds(r0, _RPT)])
    pltpu.sync_copy(zeros_hbm.at[pl.ds(r0, _RPT)], accr.at[pl.ds(r0, _RPT)])
    base, cnt = _tile_span(c * _NS + s)
    pltpu.sync_copy(src2d_hbm.at[pl.ds(base, _TMAX)], sidx2d)
    pltpu.sync_copy(dst2d_hbm.at[pl.ds(base, _TMAX)], didx2d)
    plsc.subcore_barrier()
    _ring(gl_hbm, accl, rows, gsem, ssem, sidx2d, didx2d, cnt, _NB2)
    _ring(gr_hbm, accr, rows, gsem, ssem, sidx2d, didx2d, cnt, _NB2)
    plsc.subcore_barrier()
    pltpu.sync_copy(accl.at[pl.ds(r0, _RPT)], outl_hbm.at[c, pl.ds(r0, _RPT)])
    pltpu.sync_copy(accr.at[pl.ds(r0, _RPT)], outr_hbm.at[c, pl.ds(r0, _RPT)])


_BR = 2000
_DBR = 1000


def _b1_body(x_ref, w_ref, d0_ref, d1_ref, gl_ref, gr_ref, dinv_ref):
    deg = d0_ref[...] + d1_ref[...] + 1.0
    dinv = lax.rsqrt(deg)
    h = jnp.dot(x_ref[...], w_ref[...], preferred_element_type=jnp.float32)
    g = h * dinv
    gl_ref[...] = g[:, :64]
    gr_ref[...] = g[:, 64:]
    dinv_ref[...] = dinv


def _b1(x, w1, d0, d1):
    nb = _N // _BR
    return pl.pallas_call(
        _b1_body,
        grid=(nb,),
        in_specs=[
            pl.BlockSpec((_BR, 128), lambda i: (i, 0)),
            pl.BlockSpec((128, 128), lambda i: (0, 0)),
            pl.BlockSpec((_BR, 1), lambda i: (i, 0)),
            pl.BlockSpec((_BR, 1), lambda i: (i, 0)),
        ],
        out_specs=[
            pl.BlockSpec((_BR, 64), lambda i: (i, 0)),
            pl.BlockSpec((_BR, 64), lambda i: (i, 0)),
            pl.BlockSpec((_BR, 1), lambda i: (i, 0)),
        ],
        out_shape=[
            jax.ShapeDtypeStruct((_N, 64), jnp.float32),
            jax.ShapeDtypeStruct((_N, 64), jnp.float32),
            jax.ShapeDtypeStruct((_N, 1), jnp.float32),
        ],
    )(x, w1, d0, d1)


def _b2_body(a0l_ref, a1l_ref, a0r_ref, a1r_ref, gl_ref, gr_ref, dinv_ref,
             bl_ref, br_ref, wl_ref, wr_ref, out_ref):
    dinv = dinv_ref[...]
    hl = jnp.maximum((a0l_ref[...] + a1l_ref[...] + gl_ref[...]) * dinv + bl_ref[...], 0.0)
    hr = jnp.maximum((a0r_ref[...] + a1r_ref[...] + gr_ref[...]) * dinv + br_ref[...], 0.0)
    h = (jnp.dot(hl, wl_ref[...], preferred_element_type=jnp.float32)
         + jnp.dot(hr, wr_ref[...], preferred_element_type=jnp.float32))
    out_ref[...] = h * dinv


def _b2(a0l, a1l, a0r, a1r, gl, gr, dinv, b1v, w2):
    nb = _N // _BR
    row = pl.BlockSpec((_BR, 64), lambda i: (i, 0))
    return pl.pallas_call(
        _b2_body,
        grid=(nb,),
        in_specs=[
            row, row, row, row, row, row,
            pl.BlockSpec((_BR, 1), lambda i: (i, 0)),
            pl.BlockSpec((1, 64), lambda i: (0, 0)),
            pl.BlockSpec((1, 64), lambda i: (0, 0)),
            pl.BlockSpec((64, 64), lambda i: (0, 0)),
            pl.BlockSpec((64, 64), lambda i: (0, 0)),
        ],
        out_specs=row,
        out_shape=jax.ShapeDtypeStruct((_N, 64), jnp.float32),
    )(a0l, a1l, a0r, a1r, gl, gr, dinv,
      b1v[:64].reshape(1, 64), b1v[64:].reshape(1, 64),
      w2[:64], w2[64:])


def _b3_body(a0_ref, a1_ref, g_ref, dinv_ref, b_ref, w_ref, out_ref):
    pre = (a0_ref[...] + a1_ref[...] + g_ref[...]) * dinv_ref[...] + b_ref[...]
    h = jnp.maximum(pre, 0.0)
    out_ref[...] = jnp.dot(h, w_ref[...], preferred_element_type=jnp.float32) * dinv_ref[...]


def _b3(a0, a1, g, dinv, b, w):
    nb = _N // _BR
    row = pl.BlockSpec((_BR, 64), lambda i: (i, 0))
    return pl.pallas_call(
        _b3_body,
        grid=(nb,),
        in_specs=[
            row, row, row,
            pl.BlockSpec((_BR, 1), lambda i: (i, 0)),
            pl.BlockSpec((1, 64), lambda i: (0, 0)),
            pl.BlockSpec((64, 64), lambda i: (0, 0)),
        ],
        out_specs=row,
        out_shape=jax.ShapeDtypeStruct((_N, 64), jnp.float32),
    )(a0, a1, g, dinv, b, w)


def _pool_body(a0_ref, a1_ref, g_ref, dinv_ref, b_ref, batch_ref, wf_ref, bf_ref,
               out_ref, sums, cnt):
    i = pl.program_id(0)
    nb = pl.num_programs(0)
    h = (a0_ref[...] + a1_ref[...] + g_ref[...]) * dinv_ref[...] + b_ref[...]
    m = (batch_ref[...] == lax.broadcasted_iota(jnp.int32, (_DBR, _G), 1)
         ).astype(jnp.float32)
    ssum = lax.dot_general(m, h, (((0,), (0,)), ((), ())),
                           preferred_element_type=jnp.float32)
    csum = lax.dot_general(m, jnp.ones((_DBR, 1), jnp.float32),
                           (((0,), (0,)), ((), ())),
                           preferred_element_type=jnp.float32)

    @pl.when(i == 0)
    def _():
        sums[...] = jnp.zeros_like(sums)
        cnt[...] = jnp.zeros_like(cnt)

    sums[...] += ssum
    cnt[...] += csum

    @pl.when(i == nb - 1)
    def _():
        pooled = sums[...] / jnp.maximum(cnt[...], 1.0)
        sg = jax.nn.sigmoid(pooled)
        out_ref[...] = jnp.dot(sg, wf_ref[...],
                               preferred_element_type=jnp.float32) + bf_ref[...]


def _pool(a0, a1, g, dinv, b, batch2d, wf, bf2d):
    nb = _N // _DBR
    return pl.pallas_call(
        _pool_body,
        grid=(nb,),
        in_specs=[
            pl.BlockSpec((_DBR, 64), lambda i: (i, 0)),
            pl.BlockSpec((_DBR, 64), lambda i: (i, 0)),
            pl.BlockSpec((_DBR, 64), lambda i: (i, 0)),
            pl.BlockSpec((_DBR, 1), lambda i: (i, 0)),
            pl.BlockSpec((1, 64), lambda i: (0, 0)),
            pl.BlockSpec((_DBR, 1), lambda i: (i, 0)),
            pl.BlockSpec((64, 1), lambda i: (0, 0)),
            pl.BlockSpec((1, 1), lambda i: (0, 0)),
        ],
        out_specs=pl.BlockSpec((_G, 1), lambda i: (0, 0)),
        out_shape=jax.ShapeDtypeStruct((_G, 1), jnp.float32),
        scratch_shapes=[
            pltpu.VMEM((_G, 64), jnp.float32),
            pltpu.VMEM((_G, 1), jnp.float32),
        ],
    )(a0, a1, g, dinv, b, batch2d, wf, bf2d)


def kernel(x, edge_index, batch, W1, b1, W2, b2, W3, b3, Wf, bf):
    pad = jnp.zeros((_NCHPAD * _CHUNK - _E,), jnp.int32)
    src2d = jnp.concatenate([edge_index[0], pad]).reshape(_NCHPAD, _CHUNK)
    dst2d = jnp.concatenate([edge_index[1], pad]).reshape(_NCHPAD, _CHUNK)

    zeros64 = jnp.zeros((_NPAD, 64), jnp.float32)
    zerosw = jnp.zeros((_NPAD, _DEGW), jnp.float32)
    ones_chunk = jnp.ones((_CHUNK, _DEGW), jnp.float32)

    degp = _deg_sc(dst2d, ones_chunk, zerosw)
    d0 = degp[0, :_N, 0:1]
    d1 = degp[1, :_N, 0:1]

    gl, gr, dinv = _b1(x, W1, d0, d1)
    a1l, a1r = _agg_l1(gl, gr, src2d, dst2d, zeros64)
    g2 = _b2(a1l[0, :_N], a1l[1, :_N], a1r[0, :_N], a1r[1, :_N],
             gl, gr, dinv, b1, W2)
    a2 = _agg64(g2, src2d, dst2d, zeros64)
    g3 = _b3(a2[0, :_N], a2[1, :_N], g2, dinv,
             b2.reshape(1, 64), W3)
    a3 = _agg64(g3, src2d, dst2d, zeros64)
    out = _pool(a3[0, :_N], a3[1, :_N], g3, dinv,
                b3.reshape(1, 64), batch.reshape(_N, 1), Wf, bf.reshape(1, 1))
    return out


# speedup vs baseline: 1.0694x; 1.0694x over previous
import functools

import jax
import jax.numpy as jnp
from jax import lax
from jax.experimental import pallas as pl
from jax.experimental.pallas import tpu as pltpu
from jax.experimental.pallas import tpu_sc as plsc

_N = 10000
_E = 320000
_G = 64

_NC = 2
_NS = 16
_NPAD = 10112
_RPT = _NPAD // _NS
_CHUNK = 128
_NCHUNKS = _E // _CHUNK
_CH_PER_CORE = _NCHUNKS // _NC
_T_STEPS = -(-_CH_PER_CORE // _NS)

_mesh = plsc.VectorSubcoreMesh(core_axis_name="c", subcore_axis_name="s")

_TMAX = 79
_NCHPAD = 2528
_NB = 4


_DEGW = 16


@functools.partial(
    pl.kernel,
    out_type=jax.ShapeDtypeStruct((_NC, _NPAD, _DEGW), jnp.float32),
    mesh=_mesh,
    scratch_types=[
        pltpu.VMEM_SHARED((_NPAD, _DEGW), jnp.float32),
        pltpu.VMEM((_CHUNK, _DEGW), jnp.float32),
        pltpu.VMEM((_TMAX, _CHUNK), jnp.int32),
        pltpu.SemaphoreType.DMA,
    ],
    name="gcn_deg",
    compiler_params=pltpu.CompilerParams(use_tc_tiling_on_sc=False),
)
def _deg_sc(dst2d_hbm, ones_hbm, zeros_hbm, out_hbm, acc, ones_v, didx2d, sem):
    c = lax.axis_index("c")
    s = lax.axis_index("s")
    w = c * _NS + s
    r0 = s * _RPT
    pltpu.sync_copy(zeros_hbm.at[pl.ds(r0, _RPT)], acc.at[pl.ds(r0, _RPT)])
    pltpu.sync_copy(ones_hbm, ones_v)
    pltpu.sync_copy(dst2d_hbm.at[pl.ds(78 * w, 78)], didx2d.at[pl.ds(0, 78)])

    @pl.when(w < 4)
    def _():
        pltpu.sync_copy(dst2d_hbm.at[pl.ds(2496 + w, 1)], didx2d.at[pl.ds(78, 1)])

    cnt = jnp.where(w < 4, 79, 78)
    plsc.subcore_barrier()

    def body(grp, carry):
        for b in range(_NB):
            t = grp * _NB + b

            @pl.when(t < cnt)
            def _():
                pltpu.async_copy(ones_v, acc.at[didx2d.at[t]], sem, add=True)

        for b in range(_NB):
            t = grp * _NB + b

            @pl.when(t < cnt)
            def _():
                pltpu.make_async_copy(ones_v, acc.at[didx2d.at[0]], sem).wait()

        return carry

    lax.fori_loop(0, -(-_TMAX // _NB), body, 0)
    plsc.subcore_barrier()
    pltpu.sync_copy(acc.at[pl.ds(r0, _RPT)], out_hbm.at[c, pl.ds(r0, _RPT)])


def _ring(g_hbm, acc, rows, gsem, ssem, sidx2d, didx2d, cnt, nb):
    def issue_gather(b, t):
        pltpu.async_copy(g_hbm.at[sidx2d.at[t]], rows[b], gsem[b])

    def wait_gather(b):
        pltpu.make_async_copy(g_hbm.at[sidx2d.at[0]], rows[b], gsem[b]).wait()

    def issue_scat(b, t):
        pltpu.async_copy(rows[b], acc.at[didx2d.at[t]], ssem[b], add=True)

    def wait_scat(b):
        pltpu.make_async_copy(rows[b], acc.at[didx2d.at[0]], ssem[b]).wait()

    for b in range(nb):
        issue_gather(b, b)

    def body(grp, carry):
        for b in range(nb):
            t = grp * nb + b

            @pl.when(t < cnt)
            def _():
                wait_gather(b)
                issue_scat(b, t)

        for b in range(nb):
            t = grp * nb + b

            @pl.when(t + nb < cnt)
            def _():
                wait_scat(b)
                issue_gather(b, t + nb)

        return carry

    lax.fori_loop(0, -(-_TMAX // nb), body, 0)
    for b in range(nb):
        wait_scat(b)


def _load_idx_slabs(src2d_hbm, dst2d_hbm, sidx2d, didx2d, w):
    base = 78 * w
    pltpu.sync_copy(src2d_hbm.at[pl.ds(base, 78)], sidx2d.at[pl.ds(0, 78)])
    pltpu.sync_copy(dst2d_hbm.at[pl.ds(base, 78)], didx2d.at[pl.ds(0, 78)])

    @pl.when(w < 4)
    def _():
        pltpu.sync_copy(src2d_hbm.at[pl.ds(2496 + w, 1)], sidx2d.at[pl.ds(78, 1)])
        pltpu.sync_copy(dst2d_hbm.at[pl.ds(2496 + w, 1)], didx2d.at[pl.ds(78, 1)])

    return jnp.where(w < 4, 79, 78)


_NB1 = 8


@functools.partial(
    pl.kernel,
    out_type=jax.ShapeDtypeStruct((_NC, _NPAD, 64), jnp.float32),
    mesh=_mesh,
    scratch_types=(
        [pltpu.VMEM_SHARED((_NPAD, 64), jnp.float32)]
        + [pltpu.VMEM((_CHUNK, 64), jnp.float32) for _ in range(_NB1)]
        + [pltpu.VMEM((_TMAX, _CHUNK), jnp.int32),
           pltpu.VMEM((_TMAX, _CHUNK), jnp.int32)]
        + [pltpu.SemaphoreType.DMA for _ in range(2 * _NB1)]
    ),
    name="gcn_agg_f64",
    compiler_params=pltpu.CompilerParams(use_tc_tiling_on_sc=False),
)
def _agg64(g_hbm, src2d_hbm, dst2d_hbm, zeros_hbm, out_hbm, acc, *sc):
    rows = list(sc[:_NB1])
    sidx2d, didx2d = sc[_NB1], sc[_NB1 + 1]
    gsem = list(sc[_NB1 + 2:2 * _NB1 + 2])
    ssem = list(sc[2 * _NB1 + 2:])
    c = lax.axis_index("c")
    s = lax.axis_index("s")
    r0 = s * _RPT
    pltpu.sync_copy(zeros_hbm.at[pl.ds(r0, _RPT)], acc.at[pl.ds(r0, _RPT)])
    cnt = _load_idx_slabs(src2d_hbm, dst2d_hbm, sidx2d, didx2d, c * _NS + s)
    plsc.subcore_barrier()
    _ring(g_hbm, acc, rows, gsem, ssem, sidx2d, didx2d, cnt, _NB1)
    plsc.subcore_barrier()
    pltpu.sync_copy(acc.at[pl.ds(r0, _RPT)], out_hbm.at[c, pl.ds(r0, _RPT)])


_NB2 = 3


@functools.partial(
    pl.kernel,
    out_type=[jax.ShapeDtypeStruct((_NC, _NPAD, 64), jnp.float32),
              jax.ShapeDtypeStruct((_NC, _NPAD, 64), jnp.float32)],
    mesh=_mesh,
    scratch_types=(
        [pltpu.VMEM_SHARED((_NPAD, 64), jnp.float32),
         pltpu.VMEM_SHARED((_NPAD, 64), jnp.float32)]
        + [pltpu.VMEM((_CHUNK, 64), jnp.float32) for _ in range(_NB2)]
        + [pltpu.VMEM((_TMAX, _CHUNK), jnp.int32),
           pltpu.VMEM((_TMAX, _CHUNK), jnp.int32)]
        + [pltpu.SemaphoreType.DMA for _ in range(2 * _NB2)]
    ),
    name="gcn_agg_l1",
    compiler_params=pltpu.CompilerParams(use_tc_tiling_on_sc=False),
)
def _agg_l1(gl_hbm, gr_hbm, src2d_hbm, dst2d_hbm, zeros_hbm,
            outl_hbm, outr_hbm, accl, accr, *sc):
    rows = list(sc[:_NB2])
    sidx2d, didx2d = sc[_NB2], sc[_NB2 + 1]
    gsem = list(sc[_NB2 + 2:2 * _NB2 + 2])
    ssem = list(sc[2 * _NB2 + 2:])
    c = lax.axis_index("c")
    s = lax.axis_index("s")
    r0 = s * _RPT
    pltpu.sync_copy(zeros_hbm.at[pl.ds(r0, _RPT)], accl.at[pl.ds(r0, _RPT)])
    pltpu.sync_copy(zeros_hbm.at[pl.ds(r0, _RPT)], accr.at[pl.ds(r0, _RPT)])
    cnt = _load_idx_slabs(src2d_hbm, dst2d_hbm, sidx2d, didx2d, c * _NS + s)
    plsc.subcore_barrier()
    _ring(gl_hbm, accl, rows, gsem, ssem, sidx2d, didx2d, cnt, _NB2)
    _ring(gr_hbm, accr, rows, gsem, ssem, sidx2d, didx2d, cnt, _NB2)
    plsc.subcore_barrier()
    pltpu.sync_copy(accl.at[pl.ds(r0, _RPT)], outl_hbm.at[c, pl.ds(r0, _RPT)])
    pltpu.sync_copy(accr.at[pl.ds(r0, _RPT)], outr_hbm.at[c, pl.ds(r0, _RPT)])


_BR = 2000
_DBR = 1000


def _b1a_body(x_ref, w_ref, hl_ref, hr_ref):
    h = jnp.dot(x_ref[...], w_ref[...], preferred_element_type=jnp.float32)
    hl_ref[...] = h[:, :64]
    hr_ref[...] = h[:, 64:]


def _b1a(x, w1):
    nb = _N // _BR
    return pl.pallas_call(
        _b1a_body,
        grid=(nb,),
        in_specs=[
            pl.BlockSpec((_BR, 128), lambda i: (i, 0)),
            pl.BlockSpec((128, 128), lambda i: (0, 0)),
        ],
        out_specs=[
            pl.BlockSpec((_BR, 64), lambda i: (i, 0)),
            pl.BlockSpec((_BR, 64), lambda i: (i, 0)),
        ],
        out_shape=[
            jax.ShapeDtypeStruct((_N, 64), jnp.float32),
            jax.ShapeDtypeStruct((_N, 64), jnp.float32),
        ],
    )(x, w1)


def _b1b_body(hl_ref, hr_ref, d0_ref, d1_ref, gl_ref, gr_ref, dinv_ref):
    deg = d0_ref[0, :, 0:1] + d1_ref[0, :, 0:1] + 1.0
    dinv = lax.rsqrt(deg)
    gl_ref[...] = hl_ref[...] * dinv
    gr_ref[...] = hr_ref[...] * dinv
    dinv_ref[...] = dinv


def _b1b(hl, hr, degp):
    nb = _N // _BR
    row = pl.BlockSpec((_BR, 64), lambda i: (i, 0))
    dcol = pl.BlockSpec((1, _BR, _DEGW), lambda i: (0, i, 0))
    dcol1 = pl.BlockSpec((1, _BR, _DEGW), lambda i: (1, i, 0))
    return pl.pallas_call(
        _b1b_body,
        grid=(nb,),
        in_specs=[row, row, dcol, dcol1],
        out_specs=[
            row, row,
            pl.BlockSpec((_BR, 1), lambda i: (i, 0)),
        ],
        out_shape=[
            jax.ShapeDtypeStruct((_N, 64), jnp.float32),
            jax.ShapeDtypeStruct((_N, 64), jnp.float32),
            jax.ShapeDtypeStruct((_N, 1), jnp.float32),
        ],
    )(hl, hr, degp, degp)


def _b2_body(a0l_ref, a1l_ref, a0r_ref, a1r_ref, gl_ref, gr_ref, dinv_ref,
             bl_ref, br_ref, wl_ref, wr_ref, out_ref):
    dinv = dinv_ref[...]
    hl = jnp.maximum((a0l_ref[0] + a1l_ref[0] + gl_ref[...]) * dinv + bl_ref[...], 0.0)
    hr = jnp.maximum((a0r_ref[0] + a1r_ref[0] + gr_ref[...]) * dinv + br_ref[...], 0.0)
    h = (jnp.dot(hl, wl_ref[...], preferred_element_type=jnp.float32)
         + jnp.dot(hr, wr_ref[...], preferred_element_type=jnp.float32))
    out_ref[...] = h * dinv


def _b2(a1l, a1r, gl, gr, dinv, b1v, w2):
    nb = _N // _BR
    row = pl.BlockSpec((_BR, 64), lambda i: (i, 0))
    p0 = pl.BlockSpec((1, _BR, 64), lambda i: (0, i, 0))
    p1 = pl.BlockSpec((1, _BR, 64), lambda i: (1, i, 0))
    return pl.pallas_call(
        _b2_body,
        grid=(nb,),
        in_specs=[
            p0, p1, p0, p1, row, row,
            pl.BlockSpec((_BR, 1), lambda i: (i, 0)),
            pl.BlockSpec((1, 64), lambda i: (0, 0)),
            pl.BlockSpec((1, 64), lambda i: (0, 0)),
            pl.BlockSpec((64, 64), lambda i: (0, 0)),
            pl.BlockSpec((64, 64), lambda i: (0, 0)),
        ],
        out_specs=row,
        out_shape=jax.ShapeDtypeStruct((_N, 64), jnp.float32),
    )(a1l, a1l, a1r, a1r, gl, gr, dinv,
      b1v[:64].reshape(1, 64), b1v[64:].reshape(1, 64),
      w2[:64], w2[64:])


def _b3_body(a0_ref, a1_ref, g_ref, dinv_ref, b_ref, w_ref, out_ref):
    pre = (a0_ref[0] + a1_ref[0] + g_ref[...]) * dinv_ref[...] + b_ref[...]
    h = jnp.maximum(pre, 0.0)
    out_ref[...] = jnp.dot(h, w_ref[...], preferred_element_type=jnp.float32) * dinv_ref[...]


def _b3(a2, g, dinv, b, w):
    nb = _N // _BR
    row = pl.BlockSpec((_BR, 64), lambda i: (i, 0))
    p0 = pl.BlockSpec((1, _BR, 64), lambda i: (0, i, 0))
    p1 = pl.BlockSpec((1, _BR, 64), lambda i: (1, i, 0))
    return pl.pallas_call(
        _b3_body,
        grid=(nb,),
        in_specs=[
            p0, p1, row,
            pl.BlockSpec((_BR, 1), lambda i: (i, 0)),
            pl.BlockSpec((1, 64), lambda i: (0, 0)),
            pl.BlockSpec((64, 64), lambda i: (0, 0)),
        ],
        out_specs=row,
        out_shape=jax.ShapeDtypeStruct((_N, 64), jnp.float32),
    )(a2, a2, g, dinv, b, w)


def _pool_body(a0_ref, a1_ref, g_ref, dinv_ref, b_ref, batch_ref, wf_ref, bf_ref,
               out_ref, sums, cnt):
    i = pl.program_id(0)
    nb = pl.num_programs(0)
    h = (a0_ref[0] + a1_ref[0] + g_ref[...]) * dinv_ref[...] + b_ref[...]
    m = (batch_ref[...] == lax.broadcasted_iota(jnp.int32, (_DBR, _G), 1)
         ).astype(jnp.float32)
    ssum = lax.dot_general(m, h, (((0,), (0,)), ((), ())),
                           preferred_element_type=jnp.float32)
    csum = lax.dot_general(m, jnp.ones((_DBR, 1), jnp.float32),
                           (((0,), (0,)), ((), ())),
                           preferred_element_type=jnp.float32)

    @pl.when(i == 0)
    def _():
        sums[...] = jnp.zeros_like(sums)
        cnt[...] = jnp.zeros_like(cnt)

    sums[...] += ssum
    cnt[...] += csum

    @pl.when(i == nb - 1)
    def _():
        pooled = sums[...] / jnp.maximum(cnt[...], 1.0)
        sg = jax.nn.sigmoid(pooled)
        out_ref[...] = jnp.dot(sg, wf_ref[...],
                               preferred_element_type=jnp.float32) + bf_ref[...]


def _pool(a3, g, dinv, b, batch2d, wf, bf2d):
    nb = _N // _DBR
    return pl.pallas_call(
        _pool_body,
        grid=(nb,),
        in_specs=[
            pl.BlockSpec((1, _DBR, 64), lambda i: (0, i, 0)),
            pl.BlockSpec((1, _DBR, 64), lambda i: (1, i, 0)),
            pl.BlockSpec((_DBR, 64), lambda i: (i, 0)),
            pl.BlockSpec((_DBR, 1), lambda i: (i, 0)),
            pl.BlockSpec((1, 64), lambda i: (0, 0)),
            pl.BlockSpec((_DBR, 1), lambda i: (i, 0)),
            pl.BlockSpec((64, 1), lambda i: (0, 0)),
            pl.BlockSpec((1, 1), lambda i: (0, 0)),
        ],
        out_specs=pl.BlockSpec((_G, 1), lambda i: (0, 0)),
        out_shape=jax.ShapeDtypeStruct((_G, 1), jnp.float32),
        scratch_shapes=[
            pltpu.VMEM((_G, 64), jnp.float32),
            pltpu.VMEM((_G, 1), jnp.float32),
        ],
    )(a3, a3, g, dinv, b, batch2d, wf, bf2d)


def kernel(x, edge_index, batch, W1, b1, W2, b2, W3, b3, Wf, bf):
    src2d = edge_index[0].reshape(_NCHUNKS, _CHUNK)
    dst2d = edge_index[1].reshape(_NCHUNKS, _CHUNK)

    zeros64 = jnp.zeros((_NPAD, 64), jnp.float32)
    zerosw = jnp.zeros((_NPAD, _DEGW), jnp.float32)
    ones_chunk = jnp.ones((_CHUNK, _DEGW), jnp.float32)

    degp = _deg_sc(dst2d, ones_chunk, zerosw)
    hl, hr = _b1a(x, W1)
    gl, gr, dinv = _b1b(hl, hr, degp)
    a1l, a1r = _agg_l1(gl, gr, src2d, dst2d, zeros64)
    g2 = _b2(a1l, a1r, gl, gr, dinv, b1, W2)
    a2 = _agg64(g2, src2d, dst2d, zeros64)
    g3 = _b3(a2, g2, dinv, b2.reshape(1, 64), W3)
    a3 = _agg64(g3, src2d, dst2d, zeros64)
    out = _pool(a3, g3, dinv, b3.reshape(1, 64),
                batch.reshape(_N, 1), Wf, bf.reshape(1, 1))
    return out


# speedup vs baseline: 1.0940x; 1.0230x over previous
import functools

import jax
import jax.numpy as jnp
from jax import lax
from jax.experimental import pallas as pl
from jax.experimental.pallas import tpu as pltpu
from jax.experimental.pallas import tpu_sc as plsc

_N = 10000
_E = 320000
_G = 64

_NC = 2
_NS = 16
_NPAD = 10112
_RPT = _NPAD // _NS
_CHUNK = 128
_NCHUNKS = _E // _CHUNK
_CH_PER_CORE = _NCHUNKS // _NC
_T_STEPS = -(-_CH_PER_CORE // _NS)

_mesh = plsc.VectorSubcoreMesh(core_axis_name="c", subcore_axis_name="s")

_TMAX = 79
_NCHPAD = 2528
_NB = 4


_DEGW = 16


@functools.partial(
    pl.kernel,
    out_type=jax.ShapeDtypeStruct((_NC, _NPAD, _DEGW), jnp.float32),
    mesh=_mesh,
    scratch_types=[
        pltpu.VMEM_SHARED((_NPAD, _DEGW), jnp.float32),
        pltpu.VMEM((_CHUNK, _DEGW), jnp.float32),
        pltpu.VMEM((_TMAX, _CHUNK), jnp.int32),
        pltpu.SemaphoreType.DMA,
    ],
    name="gcn_deg",
    compiler_params=pltpu.CompilerParams(use_tc_tiling_on_sc=False),
)
def _deg_sc(dst2d_hbm, ones_hbm, zeros_hbm, out_hbm, acc, ones_v, didx2d, sem):
    c = lax.axis_index("c")
    s = lax.axis_index("s")
    w = c * _NS + s
    r0 = s * _RPT
    z0 = pltpu.async_copy(zeros_hbm.at[pl.ds(r0, _RPT)], acc.at[pl.ds(r0, _RPT)], sem)
    o0 = pltpu.async_copy(ones_hbm, ones_v, sem)
    d0 = pltpu.async_copy(dst2d_hbm.at[pl.ds(78 * w, 78)], didx2d.at[pl.ds(0, 78)], sem)

    @pl.when(w < 4)
    def _():
        pltpu.sync_copy(dst2d_hbm.at[pl.ds(2496 + w, 1)], didx2d.at[pl.ds(78, 1)])

    cnt = jnp.where(w < 4, 79, 78)
    z0.wait()
    o0.wait()
    d0.wait()
    plsc.subcore_barrier()

    def body(grp, carry):
        for b in range(_NB):
            t = grp * _NB + b

            @pl.when(t < cnt)
            def _():
                pltpu.async_copy(ones_v, acc.at[didx2d.at[t]], sem, add=True)

        for b in range(_NB):
            t = grp * _NB + b

            @pl.when(t < cnt)
            def _():
                pltpu.make_async_copy(ones_v, acc.at[didx2d.at[0]], sem).wait()

        return carry

    lax.fori_loop(0, -(-_TMAX // _NB), body, 0)
    plsc.subcore_barrier()
    pltpu.sync_copy(acc.at[pl.ds(r0, _RPT)], out_hbm.at[c, pl.ds(r0, _RPT)])


def _ring(g_hbm, acc, rows, gsem, ssem, sidx2d, didx2d, cnt, nb):
    def issue_gather(b, t):
        pltpu.async_copy(g_hbm.at[sidx2d.at[t]], rows[b], gsem[b])

    def wait_gather(b):
        pltpu.make_async_copy(g_hbm.at[sidx2d.at[0]], rows[b], gsem[b]).wait()

    def issue_scat(b, t):
        pltpu.async_copy(rows[b], acc.at[didx2d.at[t]], ssem[b], add=True)

    def wait_scat(b):
        pltpu.make_async_copy(rows[b], acc.at[didx2d.at[0]], ssem[b]).wait()

    for b in range(nb):
        issue_gather(b, b)

    def body(grp, carry):
        for b in range(nb):
            t = grp * nb + b

            @pl.when(t < cnt)
            def _():
                wait_gather(b)
                issue_scat(b, t)

        for b in range(nb):
            t = grp * nb + b

            @pl.when(t + nb < cnt)
            def _():
                wait_scat(b)
                issue_gather(b, t + nb)

        return carry

    lax.fori_loop(0, -(-_TMAX // nb), body, 0)
    for b in range(nb):
        wait_scat(b)


def _load_idx_slabs(src2d_hbm, dst2d_hbm, sidx2d, didx2d, w, sem0, sem1):
    base = 78 * w
    c0 = pltpu.async_copy(src2d_hbm.at[pl.ds(base, 78)], sidx2d.at[pl.ds(0, 78)], sem0)
    c1 = pltpu.async_copy(dst2d_hbm.at[pl.ds(base, 78)], didx2d.at[pl.ds(0, 78)], sem1)

    @pl.when(w < 4)
    def _():
        pltpu.sync_copy(src2d_hbm.at[pl.ds(2496 + w, 1)], sidx2d.at[pl.ds(78, 1)])
        pltpu.sync_copy(dst2d_hbm.at[pl.ds(2496 + w, 1)], didx2d.at[pl.ds(78, 1)])

    return jnp.where(w < 4, 79, 78), c0, c1


_NB1 = 8


@functools.partial(
    pl.kernel,
    out_type=jax.ShapeDtypeStruct((_NC, _NPAD, 64), jnp.float32),
    mesh=_mesh,
    scratch_types=(
        [pltpu.VMEM_SHARED((_NPAD, 64), jnp.float32)]
        + [pltpu.VMEM((_CHUNK, 64), jnp.float32) for _ in range(_NB1)]
        + [pltpu.VMEM((_TMAX, _CHUNK), jnp.int32),
           pltpu.VMEM((_TMAX, _CHUNK), jnp.int32)]
        + [pltpu.SemaphoreType.DMA for _ in range(2 * _NB1)]
    ),
    name="gcn_agg_f64",
    compiler_params=pltpu.CompilerParams(use_tc_tiling_on_sc=False),
)
def _agg64(g_hbm, src2d_hbm, dst2d_hbm, zeros_hbm, out_hbm, acc, *sc):
    rows = list(sc[:_NB1])
    sidx2d, didx2d = sc[_NB1], sc[_NB1 + 1]
    gsem = list(sc[_NB1 + 2:2 * _NB1 + 2])
    ssem = list(sc[2 * _NB1 + 2:])
    c = lax.axis_index("c")
    s = lax.axis_index("s")
    r0 = s * _RPT
    z0 = pltpu.async_copy(zeros_hbm.at[pl.ds(r0, _RPT)], acc.at[pl.ds(r0, _RPT)], ssem[0])
    cnt, c0, c1 = _load_idx_slabs(src2d_hbm, dst2d_hbm, sidx2d, didx2d,
                                  c * _NS + s, gsem[0], gsem[1])
    z0.wait()
    c0.wait()
    c1.wait()
    plsc.subcore_barrier()
    _ring(g_hbm, acc, rows, gsem, ssem, sidx2d, didx2d, cnt, _NB1)
    plsc.subcore_barrier()
    pltpu.sync_copy(acc.at[pl.ds(r0, _RPT)], out_hbm.at[c, pl.ds(r0, _RPT)])


_NB2 = 3


@functools.partial(
    pl.kernel,
    out_type=[jax.ShapeDtypeStruct((_NC, _NPAD, 64), jnp.float32),
              jax.ShapeDtypeStruct((_NC, _NPAD, 64), jnp.float32)],
    mesh=_mesh,
    scratch_types=(
        [pltpu.VMEM_SHARED((_NPAD, 64), jnp.float32),
         pltpu.VMEM_SHARED((_NPAD, 64), jnp.float32)]
        + [pltpu.VMEM((_CHUNK, 64), jnp.float32) for _ in range(_NB2)]
        + [pltpu.VMEM((_TMAX, _CHUNK), jnp.int32),
           pltpu.VMEM((_TMAX, _CHUNK), jnp.int32)]
        + [pltpu.SemaphoreType.DMA for _ in range(2 * _NB2)]
    ),
    name="gcn_agg_l1",
    compiler_params=pltpu.CompilerParams(use_tc_tiling_on_sc=False),
)
def _agg_l1(gl_hbm, gr_hbm, src2d_hbm, dst2d_hbm, zeros_hbm,
            outl_hbm, outr_hbm, accl, accr, *sc):
    rows = list(sc[:_NB2])
    sidx2d, didx2d = sc[_NB2], sc[_NB2 + 1]
    gsem = list(sc[_NB2 + 2:2 * _NB2 + 2])
    ssem = list(sc[2 * _NB2 + 2:])
    c = lax.axis_index("c")
    s = lax.axis_index("s")
    r0 = s * _RPT
    z0 = pltpu.async_copy(zeros_hbm.at[pl.ds(r0, _RPT)], accl.at[pl.ds(r0, _RPT)], ssem[0])
    z1 = pltpu.async_copy(zeros_hbm.at[pl.ds(r0, _RPT)], accr.at[pl.ds(r0, _RPT)], ssem[1])
    cnt, c0, c1 = _load_idx_slabs(src2d_hbm, dst2d_hbm, sidx2d, didx2d,
                                  c * _NS + s, gsem[0], gsem[1])
    z0.wait()
    z1.wait()
    c0.wait()
    c1.wait()
    plsc.subcore_barrier()
    _ring(gl_hbm, accl, rows, gsem, ssem, sidx2d, didx2d, cnt, _NB2)
    _ring(gr_hbm, accr, rows, gsem, ssem, sidx2d, didx2d, cnt, _NB2)
    plsc.subcore_barrier()
    w0 = pltpu.async_copy(accl.at[pl.ds(r0, _RPT)], outl_hbm.at[c, pl.ds(r0, _RPT)], gsem[0])
    w1 = pltpu.async_copy(accr.at[pl.ds(r0, _RPT)], outr_hbm.at[c, pl.ds(r0, _RPT)], gsem[1])
    w0.wait()
    w1.wait()


_BR = 2000
_DBR = 1000


def _b1a_body(x_ref, w_ref, hl_ref, hr_ref):
    h = jnp.dot(x_ref[...], w_ref[...], preferred_element_type=jnp.float32)
    hl_ref[...] = h[:, :64]
    hr_ref[...] = h[:, 64:]


def _b1a(x, w1):
    nb = _N // _BR
    return pl.pallas_call(
        _b1a_body,
        grid=(nb,),
        in_specs=[
            pl.BlockSpec((_BR, 128), lambda i: (i, 0)),
            pl.BlockSpec((128, 128), lambda i: (0, 0)),
        ],
        out_specs=[
            pl.BlockSpec((_BR, 64), lambda i: (i, 0)),
            pl.BlockSpec((_BR, 64), lambda i: (i, 0)),
        ],
        out_shape=[
            jax.ShapeDtypeStruct((_N, 64), jnp.float32),
            jax.ShapeDtypeStruct((_N, 64), jnp.float32),
        ],
    )(x, w1)


def _b1b_body(hl_ref, hr_ref, d0_ref, d1_ref, gl_ref, gr_ref, dinv_ref):
    deg = d0_ref[0, :, 0:1] + d1_ref[0, :, 0:1] + 1.0
    dinv = lax.rsqrt(deg)
    gl_ref[...] = hl_ref[...] * dinv
    gr_ref[...] = hr_ref[...] * dinv
    dinv_ref[...] = dinv


def _b1b(hl, hr, degp):
    nb = _N // _BR
    row = pl.BlockSpec((_BR, 64), lambda i: (i, 0))
    dcol = pl.BlockSpec((1, _BR, _DEGW), lambda i: (0, i, 0))
    dcol1 = pl.BlockSpec((1, _BR, _DEGW), lambda i: (1, i, 0))
    return pl.pallas_call(
        _b1b_body,
        grid=(nb,),
        in_specs=[row, row, dcol, dcol1],
        out_specs=[
            row, row,
            pl.BlockSpec((_BR, 1), lambda i: (i, 0)),
        ],
        out_shape=[
            jax.ShapeDtypeStruct((_N, 64), jnp.float32),
            jax.ShapeDtypeStruct((_N, 64), jnp.float32),
            jax.ShapeDtypeStruct((_N, 1), jnp.float32),
        ],
    )(hl, hr, degp, degp)


def _b2_body(a0l_ref, a1l_ref, a0r_ref, a1r_ref, gl_ref, gr_ref, dinv_ref,
             bl_ref, br_ref, wl_ref, wr_ref, out_ref):
    dinv = dinv_ref[...]
    hl = jnp.maximum((a0l_ref[0] + a1l_ref[0] + gl_ref[...]) * dinv + bl_ref[...], 0.0)
    hr = jnp.maximum((a0r_ref[0] + a1r_ref[0] + gr_ref[...]) * dinv + br_ref[...], 0.0)
    h = (jnp.dot(hl, wl_ref[...], preferred_element_type=jnp.float32)
         + jnp.dot(hr, wr_ref[...], preferred_element_type=jnp.float32))
    out_ref[...] = h * dinv


def _b2(a1l, a1r, gl, gr, dinv, b1v, w2):
    nb = _N // _BR
    row = pl.BlockSpec((_BR, 64), lambda i: (i, 0))
    p0 = pl.BlockSpec((1, _BR, 64), lambda i: (0, i, 0))
    p1 = pl.BlockSpec((1, _BR, 64), lambda i: (1, i, 0))
    return pl.pallas_call(
        _b2_body,
        grid=(nb,),
        in_specs=[
            p0, p1, p0, p1, row, row,
            pl.BlockSpec((_BR, 1), lambda i: (i, 0)),
            pl.BlockSpec((1, 64), lambda i: (0, 0)),
            pl.BlockSpec((1, 64), lambda i: (0, 0)),
            pl.BlockSpec((64, 64), lambda i: (0, 0)),
            pl.BlockSpec((64, 64), lambda i: (0, 0)),
        ],
        out_specs=row,
        out_shape=jax.ShapeDtypeStruct((_N, 64), jnp.float32),
    )(a1l, a1l, a1r, a1r, gl, gr, dinv,
      b1v[:64].reshape(1, 64), b1v[64:].reshape(1, 64),
      w2[:64], w2[64:])


def _b3_body(a0_ref, a1_ref, g_ref, dinv_ref, b_ref, w_ref, out_ref):
    pre = (a0_ref[0] + a1_ref[0] + g_ref[...]) * dinv_ref[...] + b_ref[...]
    h = jnp.maximum(pre, 0.0)
    out_ref[...] = jnp.dot(h, w_ref[...], preferred_element_type=jnp.float32) * dinv_ref[...]


def _b3(a2, g, dinv, b, w):
    nb = _N // _BR
    row = pl.BlockSpec((_BR, 64), lambda i: (i, 0))
    p0 = pl.BlockSpec((1, _BR, 64), lambda i: (0, i, 0))
    p1 = pl.BlockSpec((1, _BR, 64), lambda i: (1, i, 0))
    return pl.pallas_call(
        _b3_body,
        grid=(nb,),
        in_specs=[
            p0, p1, row,
            pl.BlockSpec((_BR, 1), lambda i: (i, 0)),
            pl.BlockSpec((1, 64), lambda i: (0, 0)),
            pl.BlockSpec((64, 64), lambda i: (0, 0)),
        ],
        out_specs=row,
        out_shape=jax.ShapeDtypeStruct((_N, 64), jnp.float32),
    )(a2, a2, g, dinv, b, w)


def _pool_body(a0_ref, a1_ref, g_ref, dinv_ref, b_ref, batch_ref, wf_ref, bf_ref,
               out_ref, sums, cnt):
    i = pl.program_id(0)
    nb = pl.num_programs(0)
    h = (a0_ref[0] + a1_ref[0] + g_ref[...]) * dinv_ref[...] + b_ref[...]
    m = (batch_ref[...] == lax.broadcasted_iota(jnp.int32, (_DBR, _G), 1)
         ).astype(jnp.float32)
    ssum = lax.dot_general(m, h, (((0,), (0,)), ((), ())),
                           preferred_element_type=jnp.float32)
    csum = lax.dot_general(m, jnp.ones((_DBR, 1), jnp.float32),
                           (((0,), (0,)), ((), ())),
                           preferred_element_type=jnp.float32)

    @pl.when(i == 0)
    def _():
        sums[...] = jnp.zeros_like(sums)
        cnt[...] = jnp.zeros_like(cnt)

    sums[...] += ssum
    cnt[...] += csum

    @pl.when(i == nb - 1)
    def _():
        pooled = sums[...] / jnp.maximum(cnt[...], 1.0)
        sg = jax.nn.sigmoid(pooled)
        out_ref[...] = jnp.dot(sg, wf_ref[...],
                               preferred_element_type=jnp.float32) + bf_ref[...]


def _pool(a3, g, dinv, b, batch2d, wf, bf2d):
    nb = _N // _DBR
    return pl.pallas_call(
        _pool_body,
        grid=(nb,),
        in_specs=[
            pl.BlockSpec((1, _DBR, 64), lambda i: (0, i, 0)),
            pl.BlockSpec((1, _DBR, 64), lambda i: (1, i, 0)),
            pl.BlockSpec((_DBR, 64), lambda i: (i, 0)),
            pl.BlockSpec((_DBR, 1), lambda i: (i, 0)),
            pl.BlockSpec((1, 64), lambda i: (0, 0)),
            pl.BlockSpec((_DBR, 1), lambda i: (i, 0)),
            pl.BlockSpec((64, 1), lambda i: (0, 0)),
            pl.BlockSpec((1, 1), lambda i: (0, 0)),
        ],
        out_specs=pl.BlockSpec((_G, 1), lambda i: (0, 0)),
        out_shape=jax.ShapeDtypeStruct((_G, 1), jnp.float32),
        scratch_shapes=[
            pltpu.VMEM((_G, 64), jnp.float32),
            pltpu.VMEM((_G, 1), jnp.float32),
        ],
    )(a3, a3, g, dinv, b, batch2d, wf, bf2d)


def kernel(x, edge_index, batch, W1, b1, W2, b2, W3, b3, Wf, bf):
    src2d = edge_index[0].reshape(_NCHUNKS, _CHUNK)
    dst2d = edge_index[1].reshape(_NCHUNKS, _CHUNK)

    zeros64 = jnp.zeros((_NPAD, 64), jnp.float32)
    zerosw = jnp.zeros((_NPAD, _DEGW), jnp.float32)
    ones_chunk = jnp.ones((_CHUNK, _DEGW), jnp.float32)

    degp = _deg_sc(dst2d, ones_chunk, zerosw)
    hl, hr = _b1a(x, W1)
    gl, gr, dinv = _b1b(hl, hr, degp)
    a1l, a1r = _agg_l1(gl, gr, src2d, dst2d, zeros64)
    g2 = _b2(a1l, a1r, gl, gr, dinv, b1, W2)
    a2 = _agg64(g2, src2d, dst2d, zeros64)
    g3 = _b3(a2, g2, dinv, b2.reshape(1, 64), W3)
    a3 = _agg64(g3, src2d, dst2d, zeros64)
    out = _pool(a3, g3, dinv, b3.reshape(1, 64),
                batch.reshape(_N, 1), Wf, bf.reshape(1, 1))
    return out


# speedup vs baseline: 1.1076x; 1.0124x over previous
import functools

import jax
import jax.numpy as jnp
from jax import lax
from jax.experimental import pallas as pl
from jax.experimental.pallas import tpu as pltpu
from jax.experimental.pallas import tpu_sc as plsc

_N = 10000
_E = 320000
_G = 64

_NC = 2
_NS = 16
_NPAD = 10112
_RPT = _NPAD // _NS
_CHUNK = 128
_NCHUNKS = _E // _CHUNK
_CH_PER_CORE = _NCHUNKS // _NC
_T_STEPS = -(-_CH_PER_CORE // _NS)

_mesh = plsc.VectorSubcoreMesh(core_axis_name="c", subcore_axis_name="s")

_TMAX = 79
_NCHPAD = 2528
_NB = 4


_DEGW = 16


@functools.partial(
    pl.kernel,
    out_type=jax.ShapeDtypeStruct((_NC, _NPAD, _DEGW), jnp.float32),
    mesh=_mesh,
    scratch_types=[
        pltpu.VMEM_SHARED((_NPAD, _DEGW), jnp.float32),
        pltpu.VMEM((_CHUNK, _DEGW), jnp.float32),
        pltpu.VMEM((_TMAX, _CHUNK), jnp.int32),
        pltpu.SemaphoreType.DMA,
    ],
    name="gcn_deg",
    compiler_params=pltpu.CompilerParams(use_tc_tiling_on_sc=False),
)
def _deg_sc(dst2d_hbm, ones_hbm, zeros_hbm, out_hbm, acc, ones_v, didx2d, sem):
    c = lax.axis_index("c")
    s = lax.axis_index("s")
    w = c * _NS + s
    r0 = s * _RPT
    z0 = pltpu.async_copy(zeros_hbm.at[pl.ds(r0, _RPT)], acc.at[pl.ds(r0, _RPT)], sem)
    o0 = pltpu.async_copy(ones_hbm, ones_v, sem)
    d0 = pltpu.async_copy(dst2d_hbm.at[pl.ds(78 * w, 78)], didx2d.at[pl.ds(0, 78)], sem)

    @pl.when(w < 4)
    def _():
        pltpu.sync_copy(dst2d_hbm.at[pl.ds(2496 + w, 1)], didx2d.at[pl.ds(78, 1)])

    cnt = jnp.where(w < 4, 79, 78)
    z0.wait()
    o0.wait()
    d0.wait()
    plsc.subcore_barrier()

    def body(grp, carry):
        for b in range(_NB):
            t = grp * _NB + b

            @pl.when(t < cnt)
            def _():
                pltpu.async_copy(ones_v, acc.at[didx2d.at[t]], sem, add=True)

        for b in range(_NB):
            t = grp * _NB + b

            @pl.when(t < cnt)
            def _():
                pltpu.make_async_copy(ones_v, acc.at[didx2d.at[0]], sem).wait()

        return carry

    lax.fori_loop(0, -(-_TMAX // _NB), body, 0)
    plsc.subcore_barrier()
    pltpu.sync_copy(acc.at[pl.ds(r0, _RPT)], out_hbm.at[c, pl.ds(r0, _RPT)])


def _ring(g_hbm, acc, rows, gsem, ssem, sidx2d, didx2d, cnt, nb):
    def issue_gather(b, t):
        pltpu.async_copy(g_hbm.at[sidx2d.at[t]], rows[b], gsem[b])

    def wait_gather(b):
        pltpu.make_async_copy(g_hbm.at[sidx2d.at[0]], rows[b], gsem[b]).wait()

    def issue_scat(b, t):
        pltpu.async_copy(rows[b], acc.at[didx2d.at[t]], ssem[b], add=True)

    def wait_scat(b):
        pltpu.make_async_copy(rows[b], acc.at[didx2d.at[0]], ssem[b]).wait()

    for b in range(nb):
        issue_gather(b, b)

    def body(grp, carry):
        for b in range(nb):
            t = grp * nb + b

            @pl.when(t < cnt)
            def _():
                wait_gather(b)
                issue_scat(b, t)

        for b in range(nb):
            t = grp * nb + b

            @pl.when(t + nb < cnt)
            def _():
                wait_scat(b)
                issue_gather(b, t + nb)

        return carry

    lax.fori_loop(0, -(-_TMAX // nb), body, 0)
    for b in range(nb):
        wait_scat(b)


def _load_idx_slabs(src2d_hbm, dst2d_hbm, sidx2d, didx2d, w, sem0, sem1):
    base = 78 * w
    c0 = pltpu.async_copy(src2d_hbm.at[pl.ds(base, 78)], sidx2d.at[pl.ds(0, 78)], sem0)
    c1 = pltpu.async_copy(dst2d_hbm.at[pl.ds(base, 78)], didx2d.at[pl.ds(0, 78)], sem1)

    @pl.when(w < 4)
    def _():
        pltpu.sync_copy(src2d_hbm.at[pl.ds(2496 + w, 1)], sidx2d.at[pl.ds(78, 1)])
        pltpu.sync_copy(dst2d_hbm.at[pl.ds(2496 + w, 1)], didx2d.at[pl.ds(78, 1)])

    return jnp.where(w < 4, 79, 78), c0, c1


_NB1 = 8


@functools.partial(
    pl.kernel,
    out_type=jax.ShapeDtypeStruct((_NC, _NPAD, 64), jnp.float32),
    mesh=_mesh,
    scratch_types=(
        [pltpu.VMEM_SHARED((_NPAD, 64), jnp.float32)]
        + [pltpu.VMEM((_CHUNK, 64), jnp.float32) for _ in range(_NB1)]
        + [pltpu.VMEM((_TMAX, _CHUNK), jnp.int32),
           pltpu.VMEM((_TMAX, _CHUNK), jnp.int32)]
        + [pltpu.SemaphoreType.DMA for _ in range(2 * _NB1)]
    ),
    name="gcn_agg_f64",
    compiler_params=pltpu.CompilerParams(use_tc_tiling_on_sc=False),
)
def _agg64(g_hbm, src2d_hbm, dst2d_hbm, zeros_hbm, out_hbm, acc, *sc):
    rows = list(sc[:_NB1])
    sidx2d, didx2d = sc[_NB1], sc[_NB1 + 1]
    gsem = list(sc[_NB1 + 2:2 * _NB1 + 2])
    ssem = list(sc[2 * _NB1 + 2:])
    c = lax.axis_index("c")
    s = lax.axis_index("s")
    r0 = s * _RPT
    z0 = pltpu.async_copy(zeros_hbm.at[pl.ds(r0, _RPT)], acc.at[pl.ds(r0, _RPT)], ssem[0])
    cnt, c0, c1 = _load_idx_slabs(src2d_hbm, dst2d_hbm, sidx2d, didx2d,
                                  c * _NS + s, gsem[0], gsem[1])
    z0.wait()
    c0.wait()
    c1.wait()
    plsc.subcore_barrier()
    _ring(g_hbm, acc, rows, gsem, ssem, sidx2d, didx2d, cnt, _NB1)
    plsc.subcore_barrier()
    pltpu.sync_copy(acc.at[pl.ds(r0, _RPT)], out_hbm.at[c, pl.ds(r0, _RPT)])


_NB2 = 3


@functools.partial(
    pl.kernel,
    out_type=[jax.ShapeDtypeStruct((_NC, _NPAD, 64), jnp.float32),
              jax.ShapeDtypeStruct((_NC, _NPAD, 64), jnp.float32)],
    mesh=_mesh,
    scratch_types=(
        [pltpu.VMEM_SHARED((_NPAD, 64), jnp.float32),
         pltpu.VMEM_SHARED((_NPAD, 64), jnp.float32)]
        + [pltpu.VMEM((_CHUNK, 64), jnp.float32) for _ in range(_NB2)]
        + [pltpu.VMEM((_TMAX, _CHUNK), jnp.int32),
           pltpu.VMEM((_TMAX, _CHUNK), jnp.int32)]
        + [pltpu.SemaphoreType.DMA for _ in range(2 * _NB2)]
    ),
    name="gcn_agg_l1",
    compiler_params=pltpu.CompilerParams(use_tc_tiling_on_sc=False),
)
def _agg_l1(gl_hbm, gr_hbm, src2d_hbm, dst2d_hbm, zeros_hbm,
            outl_hbm, outr_hbm, accl, accr, *sc):
    rows = list(sc[:_NB2])
    sidx2d, didx2d = sc[_NB2], sc[_NB2 + 1]
    gsem = list(sc[_NB2 + 2:2 * _NB2 + 2])
    ssem = list(sc[2 * _NB2 + 2:])
    c = lax.axis_index("c")
    s = lax.axis_index("s")
    r0 = s * _RPT
    z0 = pltpu.async_copy(zeros_hbm.at[pl.ds(r0, _RPT)], accl.at[pl.ds(r0, _RPT)], ssem[0])
    z1 = pltpu.async_copy(zeros_hbm.at[pl.ds(r0, _RPT)], accr.at[pl.ds(r0, _RPT)], ssem[1])
    cnt, c0, c1 = _load_idx_slabs(src2d_hbm, dst2d_hbm, sidx2d, didx2d,
                                  c * _NS + s, gsem[0], gsem[1])
    z0.wait()
    z1.wait()
    c0.wait()
    c1.wait()
    plsc.subcore_barrier()
    _ring(gl_hbm, accl, rows, gsem, ssem, sidx2d, didx2d, cnt, _NB2)
    _ring(gr_hbm, accr, rows, gsem, ssem, sidx2d, didx2d, cnt, _NB2)
    plsc.subcore_barrier()
    w0 = pltpu.async_copy(accl.at[pl.ds(r0, _RPT)], outl_hbm.at[c, pl.ds(r0, _RPT)], gsem[0])
    w1 = pltpu.async_copy(accr.at[pl.ds(r0, _RPT)], outr_hbm.at[c, pl.ds(r0, _RPT)], gsem[1])
    w0.wait()
    w1.wait()


_BR = 2000
_DBR = 1000


def _b1_body(x_ref, w_ref, d0_ref, d1_ref, gl_ref, gr_ref, dinv_ref):
    deg = d0_ref[0, :, 0:1] + d1_ref[0, :, 0:1] + 1.0
    dinv = lax.rsqrt(deg)
    h = jnp.dot(x_ref[...], w_ref[...], preferred_element_type=jnp.float32)
    g = h * dinv
    gl_ref[...] = g[:, :64]
    gr_ref[...] = g[:, 64:]
    dinv_ref[...] = dinv


def _b1(x, w1, degp):
    nb = _N // _BR
    row = pl.BlockSpec((_BR, 64), lambda i: (i, 0))
    dcol = pl.BlockSpec((1, _BR, _DEGW), lambda i: (0, i, 0))
    dcol1 = pl.BlockSpec((1, _BR, _DEGW), lambda i: (1, i, 0))
    return pl.pallas_call(
        _b1_body,
        grid=(nb,),
        in_specs=[
            pl.BlockSpec((_BR, 128), lambda i: (i, 0)),
            pl.BlockSpec((128, 128), lambda i: (0, 0)),
            dcol, dcol1,
        ],
        out_specs=[
            row, row,
            pl.BlockSpec((_BR, 1), lambda i: (i, 0)),
        ],
        out_shape=[
            jax.ShapeDtypeStruct((_N, 64), jnp.float32),
            jax.ShapeDtypeStruct((_N, 64), jnp.float32),
            jax.ShapeDtypeStruct((_N, 1), jnp.float32),
        ],
    )(x, w1, degp, degp)


def _b2_body(a0l_ref, a1l_ref, a0r_ref, a1r_ref, gl_ref, gr_ref, dinv_ref,
             bl_ref, br_ref, wl_ref, wr_ref, out_ref):
    dinv = dinv_ref[...]
    hl = jnp.maximum((a0l_ref[0] + a1l_ref[0] + gl_ref[...]) * dinv + bl_ref[...], 0.0)
    hr = jnp.maximum((a0r_ref[0] + a1r_ref[0] + gr_ref[...]) * dinv + br_ref[...], 0.0)
    h = (jnp.dot(hl, wl_ref[...], preferred_element_type=jnp.float32)
         + jnp.dot(hr, wr_ref[...], preferred_element_type=jnp.float32))
    out_ref[...] = h * dinv


def _b2(a1l, a1r, gl, gr, dinv, b1v, w2):
    nb = _N // _BR
    row = pl.BlockSpec((_BR, 64), lambda i: (i, 0))
    p0 = pl.BlockSpec((1, _BR, 64), lambda i: (0, i, 0))
    p1 = pl.BlockSpec((1, _BR, 64), lambda i: (1, i, 0))
    return pl.pallas_call(
        _b2_body,
        grid=(nb,),
        in_specs=[
            p0, p1, p0, p1, row, row,
            pl.BlockSpec((_BR, 1), lambda i: (i, 0)),
            pl.BlockSpec((1, 64), lambda i: (0, 0)),
            pl.BlockSpec((1, 64), lambda i: (0, 0)),
            pl.BlockSpec((64, 64), lambda i: (0, 0)),
            pl.BlockSpec((64, 64), lambda i: (0, 0)),
        ],
        out_specs=row,
        out_shape=jax.ShapeDtypeStruct((_N, 64), jnp.float32),
    )(a1l, a1l, a1r, a1r, gl, gr, dinv,
      b1v[:64].reshape(1, 64), b1v[64:].reshape(1, 64),
      w2[:64], w2[64:])


def _b3_body(a0_ref, a1_ref, g_ref, dinv_ref, b_ref, w_ref, out_ref):
    pre = (a0_ref[0] + a1_ref[0] + g_ref[...]) * dinv_ref[...] + b_ref[...]
    h = jnp.maximum(pre, 0.0)
    out_ref[...] = jnp.dot(h, w_ref[...], preferred_element_type=jnp.float32) * dinv_ref[...]


def _b3(a2, g, dinv, b, w):
    nb = _N // _BR
    row = pl.BlockSpec((_BR, 64), lambda i: (i, 0))
    p0 = pl.BlockSpec((1, _BR, 64), lambda i: (0, i, 0))
    p1 = pl.BlockSpec((1, _BR, 64), lambda i: (1, i, 0))
    return pl.pallas_call(
        _b3_body,
        grid=(nb,),
        in_specs=[
            p0, p1, row,
            pl.BlockSpec((_BR, 1), lambda i: (i, 0)),
            pl.BlockSpec((1, 64), lambda i: (0, 0)),
            pl.BlockSpec((64, 64), lambda i: (0, 0)),
        ],
        out_specs=row,
        out_shape=jax.ShapeDtypeStruct((_N, 64), jnp.float32),
    )(a2, a2, g, dinv, b, w)


def _pool_body(a0_ref, a1_ref, g_ref, dinv_ref, b_ref, batch_ref, wf_ref, bf_ref,
               out_ref, sums, cnt):
    i = pl.program_id(0)
    nb = pl.num_programs(0)
    h = (a0_ref[0] + a1_ref[0] + g_ref[...]) * dinv_ref[...] + b_ref[...]
    m = (batch_ref[...] == lax.broadcasted_iota(jnp.int32, (_DBR, _G), 1)
         ).astype(jnp.float32)
    ssum = lax.dot_general(m, h, (((0,), (0,)), ((), ())),
                           preferred_element_type=jnp.float32)
    csum = lax.dot_general(m, jnp.ones((_DBR, 1), jnp.float32),
                           (((0,), (0,)), ((), ())),
                           preferred_element_type=jnp.float32)

    @pl.when(i == 0)
    def _():
        sums[...] = jnp.zeros_like(sums)
        cnt[...] = jnp.zeros_like(cnt)

    sums[...] += ssum
    cnt[...] += csum

    @pl.when(i == nb - 1)
    def _():
        pooled = sums[...] / jnp.maximum(cnt[...], 1.0)
        sg = jax.nn.sigmoid(pooled)
        out_ref[...] = jnp.dot(sg, wf_ref[...],
                               preferred_element_type=jnp.float32) + bf_ref[...]


def _pool(a3, g, dinv, b, batch2d, wf, bf2d):
    nb = _N // _DBR
    return pl.pallas_call(
        _pool_body,
        grid=(nb,),
        in_specs=[
            pl.BlockSpec((1, _DBR, 64), lambda i: (0, i, 0)),
            pl.BlockSpec((1, _DBR, 64), lambda i: (1, i, 0)),
            pl.BlockSpec((_DBR, 64), lambda i: (i, 0)),
            pl.BlockSpec((_DBR, 1), lambda i: (i, 0)),
            pl.BlockSpec((1, 64), lambda i: (0, 0)),
            pl.BlockSpec((_DBR, 1), lambda i: (i, 0)),
            pl.BlockSpec((64, 1), lambda i: (0, 0)),
            pl.BlockSpec((1, 1), lambda i: (0, 0)),
        ],
        out_specs=pl.BlockSpec((_G, 1), lambda i: (0, 0)),
        out_shape=jax.ShapeDtypeStruct((_G, 1), jnp.float32),
        scratch_shapes=[
            pltpu.VMEM((_G, 64), jnp.float32),
            pltpu.VMEM((_G, 1), jnp.float32),
        ],
    )(a3, a3, g, dinv, b, batch2d, wf, bf2d)


def kernel(x, edge_index, batch, W1, b1, W2, b2, W3, b3, Wf, bf):
    src2d = edge_index[0].reshape(_NCHUNKS, _CHUNK)
    dst2d = edge_index[1].reshape(_NCHUNKS, _CHUNK)

    zeros64 = jnp.zeros((_NPAD, 64), jnp.float32)
    zerosw = jnp.zeros((_NPAD, _DEGW), jnp.float32)
    ones_chunk = jnp.ones((_CHUNK, _DEGW), jnp.float32)

    degp = _deg_sc(dst2d, ones_chunk, zerosw)
    gl, gr, dinv = _b1(x, W1, degp)
    a1l, a1r = _agg_l1(gl, gr, src2d, dst2d, zeros64)
    g2 = _b2(a1l, a1r, gl, gr, dinv, b1, W2)
    a2 = _agg64(g2, src2d, dst2d, zeros64)
    g3 = _b3(a2, g2, dinv, b2.reshape(1, 64), W3)
    a3 = _agg64(g3, src2d, dst2d, zeros64)
    out = _pool(a3, g3, dinv, b3.reshape(1, 64),
                batch.reshape(_N, 1), Wf, bf.reshape(1, 1))
    return out


# speedup vs baseline: 1.1085x; 1.0008x over previous
import functools

import jax
import jax.numpy as jnp
from jax import lax
from jax.experimental import pallas as pl
from jax.experimental.pallas import tpu as pltpu
from jax.experimental.pallas import tpu_sc as plsc

_N = 10000
_E = 320000
_G = 64

_NC = 2
_NS = 16
_NPAD = 10112
_RPT = _NPAD // _NS
_CHUNK = 128
_NCHUNKS = _E // _CHUNK

_mesh = plsc.VectorSubcoreMesh(core_axis_name="c", subcore_axis_name="s")

_TMAX = 79
_DEG_GRP = 4


_DEGW = 16


@functools.partial(
    pl.kernel,
    out_type=jax.ShapeDtypeStruct((_NC, _NPAD, _DEGW), jnp.float32),
    mesh=_mesh,
    scratch_types=[
        pltpu.VMEM_SHARED((_NPAD, _DEGW), jnp.float32),
        pltpu.VMEM((_CHUNK, _DEGW), jnp.float32),
        pltpu.VMEM((_TMAX, _CHUNK), jnp.int32),
        pltpu.SemaphoreType.DMA,
    ],
    name="gcn_deg",
    compiler_params=pltpu.CompilerParams(use_tc_tiling_on_sc=False),
)
def _deg_sc(dst2d_hbm, ones_hbm, zeros_hbm, out_hbm, acc, ones_v, didx2d, sem):
    c = lax.axis_index("c")
    s = lax.axis_index("s")
    w = c * _NS + s
    r0 = s * _RPT
    z0 = pltpu.async_copy(zeros_hbm.at[pl.ds(r0, _RPT)], acc.at[pl.ds(r0, _RPT)], sem)
    o0 = pltpu.async_copy(ones_hbm, ones_v, sem)
    d0 = pltpu.async_copy(dst2d_hbm.at[pl.ds(78 * w, 78)], didx2d.at[pl.ds(0, 78)], sem)

    @pl.when(w < 4)
    def _():
        pltpu.sync_copy(dst2d_hbm.at[pl.ds(2496 + w, 1)], didx2d.at[pl.ds(78, 1)])

    cnt = jnp.where(w < 4, 79, 78)
    z0.wait()
    o0.wait()
    d0.wait()
    plsc.subcore_barrier()

    def body(grp, carry):
        for b in range(_DEG_GRP):
            t = grp * _DEG_GRP + b

            @pl.when(t < cnt)
            def _():
                pltpu.async_copy(ones_v, acc.at[didx2d.at[t]], sem, add=True)

        for b in range(_DEG_GRP):
            t = grp * _DEG_GRP + b

            @pl.when(t < cnt)
            def _():
                pltpu.make_async_copy(ones_v, acc.at[didx2d.at[0]], sem).wait()

        return carry

    lax.fori_loop(0, -(-_TMAX // _DEG_GRP), body, 0)
    plsc.subcore_barrier()
    pltpu.sync_copy(acc.at[pl.ds(r0, _RPT)], out_hbm.at[c, pl.ds(r0, _RPT)])


def _ring(g_hbm, acc, rows, gsem, ssem, sidx2d, didx2d, cnt, nb):
    def issue_gather(b, t):
        pltpu.async_copy(g_hbm.at[sidx2d.at[t]], rows[b], gsem[b])

    def wait_gather(b):
        pltpu.make_async_copy(g_hbm.at[sidx2d.at[0]], rows[b], gsem[b]).wait()

    def issue_scat(b, t):
        pltpu.async_copy(rows[b], acc.at[didx2d.at[t]], ssem[b], add=True)

    def wait_scat(b):
        pltpu.make_async_copy(rows[b], acc.at[didx2d.at[0]], ssem[b]).wait()

    for b in range(nb):
        issue_gather(b, b)

    def body(grp, carry):
        for b in range(nb):
            t = grp * nb + b

            @pl.when(t < cnt)
            def _():
                wait_gather(b)
                issue_scat(b, t)

        for b in range(nb):
            t = grp * nb + b

            @pl.when(t + nb < cnt)
            def _():
                wait_scat(b)
                issue_gather(b, t + nb)

        return carry

    lax.fori_loop(0, -(-_TMAX // nb), body, 0)
    for b in range(nb):
        wait_scat(b)


def _load_idx_slabs(src2d_hbm, dst2d_hbm, sidx2d, didx2d, w, sem0, sem1):
    base = 78 * w
    c0 = pltpu.async_copy(src2d_hbm.at[pl.ds(base, 78)], sidx2d.at[pl.ds(0, 78)], sem0)
    c1 = pltpu.async_copy(dst2d_hbm.at[pl.ds(base, 78)], didx2d.at[pl.ds(0, 78)], sem1)

    @pl.when(w < 4)
    def _():
        pltpu.sync_copy(src2d_hbm.at[pl.ds(2496 + w, 1)], sidx2d.at[pl.ds(78, 1)])
        pltpu.sync_copy(dst2d_hbm.at[pl.ds(2496 + w, 1)], didx2d.at[pl.ds(78, 1)])

    return jnp.where(w < 4, 79, 78), c0, c1


_NB1 = 8


@functools.partial(
    pl.kernel,
    out_type=jax.ShapeDtypeStruct((_NC, _NPAD, 64), jnp.float32),
    mesh=_mesh,
    scratch_types=(
        [pltpu.VMEM_SHARED((_NPAD, 64), jnp.float32)]
        + [pltpu.VMEM((_CHUNK, 64), jnp.float32) for _ in range(_NB1)]
        + [pltpu.VMEM((_TMAX, _CHUNK), jnp.int32),
           pltpu.VMEM((_TMAX, _CHUNK), jnp.int32)]
        + [pltpu.SemaphoreType.DMA for _ in range(2 * _NB1)]
    ),
    name="gcn_agg_f64",
    compiler_params=pltpu.CompilerParams(use_tc_tiling_on_sc=False),
)
def _agg64(g_hbm, src2d_hbm, dst2d_hbm, zeros_hbm, out_hbm, acc, *sc):
    rows = list(sc[:_NB1])
    sidx2d, didx2d = sc[_NB1], sc[_NB1 + 1]
    gsem = list(sc[_NB1 + 2:2 * _NB1 + 2])
    ssem = list(sc[2 * _NB1 + 2:])
    c = lax.axis_index("c")
    s = lax.axis_index("s")
    r0 = s * _RPT
    z0 = pltpu.async_copy(zeros_hbm.at[pl.ds(r0, _RPT)], acc.at[pl.ds(r0, _RPT)], ssem[0])
    cnt, c0, c1 = _load_idx_slabs(src2d_hbm, dst2d_hbm, sidx2d, didx2d,
                                  c * _NS + s, gsem[0], gsem[1])
    z0.wait()
    c0.wait()
    c1.wait()
    plsc.subcore_barrier()
    _ring(g_hbm, acc, rows, gsem, ssem, sidx2d, didx2d, cnt, _NB1)
    plsc.subcore_barrier()
    pltpu.sync_copy(acc.at[pl.ds(r0, _RPT)], out_hbm.at[c, pl.ds(r0, _RPT)])


_NB2 = 3


@functools.partial(
    pl.kernel,
    out_type=[jax.ShapeDtypeStruct((_NC, _NPAD, 64), jnp.float32),
              jax.ShapeDtypeStruct((_NC, _NPAD, 64), jnp.float32)],
    mesh=_mesh,
    scratch_types=(
        [pltpu.VMEM_SHARED((_NPAD, 64), jnp.float32),
         pltpu.VMEM_SHARED((_NPAD, 64), jnp.float32)]
        + [pltpu.VMEM((_CHUNK, 64), jnp.float32) for _ in range(_NB2)]
        + [pltpu.VMEM((_TMAX, _CHUNK), jnp.int32),
           pltpu.VMEM((_TMAX, _CHUNK), jnp.int32)]
        + [pltpu.SemaphoreType.DMA for _ in range(2 * _NB2)]
    ),
    name="gcn_agg_l1",
    compiler_params=pltpu.CompilerParams(use_tc_tiling_on_sc=False),
)
def _agg_l1(gl_hbm, gr_hbm, src2d_hbm, dst2d_hbm, zeros_hbm,
            outl_hbm, outr_hbm, accl, accr, *sc):
    rows = list(sc[:_NB2])
    sidx2d, didx2d = sc[_NB2], sc[_NB2 + 1]
    gsem = list(sc[_NB2 + 2:2 * _NB2 + 2])
    ssem = list(sc[2 * _NB2 + 2:])
    c = lax.axis_index("c")
    s = lax.axis_index("s")
    r0 = s * _RPT
    z0 = pltpu.async_copy(zeros_hbm.at[pl.ds(r0, _RPT)], accl.at[pl.ds(r0, _RPT)], ssem[0])
    z1 = pltpu.async_copy(zeros_hbm.at[pl.ds(r0, _RPT)], accr.at[pl.ds(r0, _RPT)], ssem[1])
    cnt, c0, c1 = _load_idx_slabs(src2d_hbm, dst2d_hbm, sidx2d, didx2d,
                                  c * _NS + s, gsem[0], gsem[1])
    z0.wait()
    z1.wait()
    c0.wait()
    c1.wait()
    plsc.subcore_barrier()
    _ring(gl_hbm, accl, rows, gsem, ssem, sidx2d, didx2d, cnt, _NB2)
    _ring(gr_hbm, accr, rows, gsem, ssem, sidx2d, didx2d, cnt, _NB2)
    plsc.subcore_barrier()
    w0 = pltpu.async_copy(accl.at[pl.ds(r0, _RPT)], outl_hbm.at[c, pl.ds(r0, _RPT)], gsem[0])
    w1 = pltpu.async_copy(accr.at[pl.ds(r0, _RPT)], outr_hbm.at[c, pl.ds(r0, _RPT)], gsem[1])
    w0.wait()
    w1.wait()


_BR = 2000
_DBR = 1000


def _b1_body(x_ref, w_ref, d0_ref, d1_ref, gl_ref, gr_ref, dinv_ref):
    deg = d0_ref[0, :, 0:1] + d1_ref[0, :, 0:1] + 1.0
    dinv = lax.rsqrt(deg)
    h = jnp.dot(x_ref[...], w_ref[...], preferred_element_type=jnp.float32)
    g = h * dinv
    gl_ref[...] = g[:, :64]
    gr_ref[...] = g[:, 64:]
    dinv_ref[...] = dinv


def _b1(x, w1, degp):
    nb = _N // _BR
    row = pl.BlockSpec((_BR, 64), lambda i: (i, 0))
    dcol = pl.BlockSpec((1, _BR, _DEGW), lambda i: (0, i, 0))
    dcol1 = pl.BlockSpec((1, _BR, _DEGW), lambda i: (1, i, 0))
    return pl.pallas_call(
        _b1_body,
        grid=(nb,),
        in_specs=[
            pl.BlockSpec((_BR, 128), lambda i: (i, 0)),
            pl.BlockSpec((128, 128), lambda i: (0, 0)),
            dcol, dcol1,
        ],
        out_specs=[
            row, row,
            pl.BlockSpec((_BR, 1), lambda i: (i, 0)),
        ],
        out_shape=[
            jax.ShapeDtypeStruct((_N, 64), jnp.float32),
            jax.ShapeDtypeStruct((_N, 64), jnp.float32),
            jax.ShapeDtypeStruct((_N, 1), jnp.float32),
        ],
    )(x, w1, degp, degp)


def _b2_body(a0l_ref, a1l_ref, a0r_ref, a1r_ref, gl_ref, gr_ref, dinv_ref,
             bl_ref, br_ref, wl_ref, wr_ref, out_ref):
    dinv = dinv_ref[...]
    hl = jnp.maximum((a0l_ref[0] + a1l_ref[0] + gl_ref[...]) * dinv + bl_ref[...], 0.0)
    hr = jnp.maximum((a0r_ref[0] + a1r_ref[0] + gr_ref[...]) * dinv + br_ref[...], 0.0)
    h = (jnp.dot(hl, wl_ref[...], preferred_element_type=jnp.float32)
         + jnp.dot(hr, wr_ref[...], preferred_element_type=jnp.float32))
    out_ref[...] = h * dinv


def _b2(a1l, a1r, gl, gr, dinv, b1v, w2):
    nb = _N // _BR
    row = pl.BlockSpec((_BR, 64), lambda i: (i, 0))
    p0 = pl.BlockSpec((1, _BR, 64), lambda i: (0, i, 0))
    p1 = pl.BlockSpec((1, _BR, 64), lambda i: (1, i, 0))
    return pl.pallas_call(
        _b2_body,
        grid=(nb,),
        in_specs=[
            p0, p1, p0, p1, row, row,
            pl.BlockSpec((_BR, 1), lambda i: (i, 0)),
            pl.BlockSpec((1, 64), lambda i: (0, 0)),
            pl.BlockSpec((1, 64), lambda i: (0, 0)),
            pl.BlockSpec((64, 64), lambda i: (0, 0)),
            pl.BlockSpec((64, 64), lambda i: (0, 0)),
        ],
        out_specs=row,
        out_shape=jax.ShapeDtypeStruct((_N, 64), jnp.float32),
    )(a1l, a1l, a1r, a1r, gl, gr, dinv,
      b1v[:64].reshape(1, 64), b1v[64:].reshape(1, 64),
      w2[:64], w2[64:])


def _b3_body(a0_ref, a1_ref, g_ref, dinv_ref, b_ref, w_ref, out_ref):
    pre = (a0_ref[0] + a1_ref[0] + g_ref[...]) * dinv_ref[...] + b_ref[...]
    h = jnp.maximum(pre, 0.0)
    out_ref[...] = jnp.dot(h, w_ref[...], preferred_element_type=jnp.float32) * dinv_ref[...]


def _b3(a2, g, dinv, b, w):
    nb = _N // _BR
    row = pl.BlockSpec((_BR, 64), lambda i: (i, 0))
    p0 = pl.BlockSpec((1, _BR, 64), lambda i: (0, i, 0))
    p1 = pl.BlockSpec((1, _BR, 64), lambda i: (1, i, 0))
    return pl.pallas_call(
        _b3_body,
        grid=(nb,),
        in_specs=[
            p0, p1, row,
            pl.BlockSpec((_BR, 1), lambda i: (i, 0)),
            pl.BlockSpec((1, 64), lambda i: (0, 0)),
            pl.BlockSpec((64, 64), lambda i: (0, 0)),
        ],
        out_specs=row,
        out_shape=jax.ShapeDtypeStruct((_N, 64), jnp.float32),
    )(a2, a2, g, dinv, b, w)


def _pool_body(a0_ref, a1_ref, g_ref, dinv_ref, b_ref, batch_ref, wf_ref, bf_ref,
               out_ref, sums, cnt):
    i = pl.program_id(0)
    nb = pl.num_programs(0)
    h = (a0_ref[0] + a1_ref[0] + g_ref[...]) * dinv_ref[...] + b_ref[...]
    m = (batch_ref[...] == lax.broadcasted_iota(jnp.int32, (_DBR, _G), 1)
         ).astype(jnp.float32)
    ssum = lax.dot_general(m, h, (((0,), (0,)), ((), ())),
                           preferred_element_type=jnp.float32)
    csum = lax.dot_general(m, jnp.ones((_DBR, 1), jnp.float32),
                           (((0,), (0,)), ((), ())),
                           preferred_element_type=jnp.float32)

    @pl.when(i == 0)
    def _():
        sums[...] = jnp.zeros_like(sums)
        cnt[...] = jnp.zeros_like(cnt)

    sums[...] += ssum
    cnt[...] += csum

    @pl.when(i == nb - 1)
    def _():
        pooled = sums[...] / jnp.maximum(cnt[...], 1.0)
        sg = jax.nn.sigmoid(pooled)
        out_ref[...] = jnp.dot(sg, wf_ref[...],
                               preferred_element_type=jnp.float32) + bf_ref[...]


def _pool(a3, g, dinv, b, batch2d, wf, bf2d):
    nb = _N // _DBR
    return pl.pallas_call(
        _pool_body,
        grid=(nb,),
        in_specs=[
            pl.BlockSpec((1, _DBR, 64), lambda i: (0, i, 0)),
            pl.BlockSpec((1, _DBR, 64), lambda i: (1, i, 0)),
            pl.BlockSpec((_DBR, 64), lambda i: (i, 0)),
            pl.BlockSpec((_DBR, 1), lambda i: (i, 0)),
            pl.BlockSpec((1, 64), lambda i: (0, 0)),
            pl.BlockSpec((_DBR, 1), lambda i: (i, 0)),
            pl.BlockSpec((64, 1), lambda i: (0, 0)),
            pl.BlockSpec((1, 1), lambda i: (0, 0)),
        ],
        out_specs=pl.BlockSpec((_G, 1), lambda i: (0, 0)),
        out_shape=jax.ShapeDtypeStruct((_G, 1), jnp.float32),
        scratch_shapes=[
            pltpu.VMEM((_G, 64), jnp.float32),
            pltpu.VMEM((_G, 1), jnp.float32),
        ],
    )(a3, a3, g, dinv, b, batch2d, wf, bf2d)


def kernel(x, edge_index, batch, W1, b1, W2, b2, W3, b3, Wf, bf):
    src2d = edge_index[0].reshape(_NCHUNKS, _CHUNK)
    dst2d = edge_index[1].reshape(_NCHUNKS, _CHUNK)

    zeros64 = jnp.zeros((_NPAD, 64), jnp.float32)
    zerosw = jnp.zeros((_NPAD, _DEGW), jnp.float32)
    ones_chunk = jnp.ones((_CHUNK, _DEGW), jnp.float32)

    degp = _deg_sc(dst2d, ones_chunk, zerosw)
    gl, gr, dinv = _b1(x, W1, degp)
    a1l, a1r = _agg_l1(gl, gr, src2d, dst2d, zeros64)
    g2 = _b2(a1l, a1r, gl, gr, dinv, b1, W2)
    a2 = _agg64(g2, src2d, dst2d, zeros64)
    g3 = _b3(a2, g2, dinv, b2.reshape(1, 64), W3)
    a3 = _agg64(g3, src2d, dst2d, zeros64)
    out = _pool(a3, g3, dinv, b3.reshape(1, 64),
                batch.reshape(_N, 1), Wf, bf.reshape(1, 1))
    return out
